# Initial kernel scaffold; baseline (speedup 1.0000x reference)
import jax
import jax.numpy as jnp
from jax.experimental import pallas as pl


def kernel(x_text, edge_index, edge_type, batch, domain_ids, edge_property_id, Wt, bt, dom_emb, prop_emb, W0, root0, b0, W1, root1, b1, Ws, bs, Wst, bst, Wg1, bg1, Wg2, bg2):
    raise NotImplementedError("write your pallas kernel here")


# speedup vs baseline: 8.7726x; 8.7726x over previous
import functools

import jax
import jax.numpy as jnp
from jax import lax
from jax.experimental import pallas as pl
from jax.experimental.pallas import tpu as pltpu
from jax.experimental.pallas import tpu_sc as plsc

N = 10000
E = 320000
TD = 256
H = 128
R = 8
D = 16
P = 4
B = 8

NSC = 2
NSUB = 16
LANES = 16

CB = 128
NP = 10112
SLICE = NP // NSUB
NPP = 40064
NR = 81920
EC = E // (NSC * NSUB)
PCB = 80

BN = 400
GRID = N // BN

def _mesh():
    return plsc.VectorSubcoreMesh(
        core_axis_name="c", subcore_axis_name="s",
        num_cores=NSC, num_subcores=NSUB,
    )


def _cnt_body(dstp_h, srcq_h, off_h, zrs_h, outc_h, outp_h,
              dstv, idxw, rows, zbuf, offv, s_sh):
    cid = lax.axis_index("c")
    sid = lax.axis_index("s")
    i16 = lax.broadcasted_iota(jnp.int32, (LANES,), 0)

    pltpu.sync_copy(off_h, offv)
    pltpu.sync_copy(zrs_h, zbuf)
    offvec = offv[...]

    def zero_slice():
        zb = sid * SLICE
        for o, sz in ((0, CB), (CB, CB), (2 * CB, CB), (3 * CB, CB),
                      (4 * CB, SLICE - 4 * CB)):
            pltpu.sync_copy(zbuf.at[pl.ds(0, sz)], s_sh.at[pl.ds(zb + o, sz)])

    def fill_onehot(col):
        def rb(i, _):
            for cg in range(H // LANES):
                v = jnp.where(cg * LANES + i16 == col, 1.0, 0.0)
                rows[i, pl.ds(cg * LANES, LANES)] = v
            return 0
        lax.fori_loop(0, CB, rb, 0)

    def add_range(idx_h, lo, hi):
        lo8 = jnp.bitwise_and(lo, -8)
        nch = (hi - lo8 + CB - 1) // CB
        trip = jnp.maximum(0, (nch - sid + NSUB - 1) // NSUB)

        def chunk(k, _):
            st = pl.multiple_of(lo8 + (sid + k * NSUB) * CB, 8)
            pltpu.sync_copy(idx_h.at[pl.ds(st, CB)], dstv)
            for g in range(CB // LANES):
                pos = st + g * LANES + i16
                d = dstv[pl.ds(g * LANES, LANES)]
                inb = jnp.logical_and(pos >= lo, pos < hi)
                idxw[0, pl.ds(g * LANES, LANES)] = jnp.where(inb, d, N)
            pltpu.sync_copy(rows, s_sh.at[idxw.at[0]], add=True)
            return 0

        lax.fori_loop(0, trip, chunk, 0)

    half = R // NSC
    zero_slice()
    plsc.subcore_barrier()
    for j in range(half):
        r = cid * half + j
        lo = jnp.where(cid == 0, offvec[j], offvec[half + j])
        hi = jnp.where(cid == 0, offvec[j + 1], offvec[half + j + 1])
        fill_onehot(r * LANES)
        add_range(dstp_h, lo, hi)
    plsc.subcore_barrier()
    pltpu.sync_copy(s_sh.at[pl.ds(sid * SLICE, SLICE)],
                    outc_h.at[pl.ds(cid * NP + sid * SLICE, SLICE)])
    plsc.subcore_barrier()

    phalf = P // NSC
    zero_slice()
    plsc.subcore_barrier()
    for j in range(phalf):
        p = cid * phalf + j
        lo = jnp.where(cid == 0, offvec[R + 1 + j], offvec[R + 1 + phalf + j])
        hi = jnp.where(cid == 0, offvec[R + 2 + j], offvec[R + 2 + phalf + j])
        fill_onehot(p * LANES)
        add_range(srcq_h, lo, hi)
    plsc.subcore_barrier()
    pltpu.sync_copy(s_sh.at[pl.ds(sid * SLICE, SLICE)],
                    outp_h.at[pl.ds(cid * NP + sid * SLICE, SLICE)])


def _make_cnt_call(interpret=False):
    return pl.kernel(
        _cnt_body,
        out_type=(
            jax.ShapeDtypeStruct((NSC * NP, H), jnp.float32),
            jax.ShapeDtypeStruct((NSC * NP, H), jnp.float32),
        ),
        mesh=_mesh(),
        scratch_types=[
            pltpu.VMEM((CB,), jnp.int32),
            pltpu.VMEM((1, CB), jnp.int32),
            pltpu.VMEM((CB, H), jnp.float32),
            pltpu.VMEM((CB, H), jnp.float32),
            pltpu.VMEM((LANES,), jnp.int32),
            pltpu.VMEM_SHARED((NP, H), jnp.float32),
        ],
        interpret=interpret,
    )


def _agg_body(cin_h, srcp_h, dstp_h, off_h, zrs_h,
              s_out,
              srcv, dstv, idxw, rows, zbuf, offv, s_sh, sem):
    cid = lax.axis_index("c")
    sid = lax.axis_index("s")
    i16 = lax.broadcasted_iota(jnp.int32, (LANES,), 0)

    pltpu.sync_copy(off_h, offv)
    pltpu.sync_copy(zrs_h, zbuf)
    offvec = offv[...]

    half = R // NSC
    for j in range(half):
        r = cid * half + j
        lo = jnp.where(cid == 0, offvec[j], offvec[half + j])
        hi = jnp.where(cid == 0, offvec[j + 1], offvec[half + j + 1])
        lo8 = jnp.bitwise_and(lo, -8)
        nch = (hi - lo8 + CB - 1) // CB

        zb = sid * SLICE
        for o, sz in ((0, CB), (CB, CB), (2 * CB, CB), (3 * CB, CB),
                      (4 * CB, SLICE - 4 * CB)):
            pltpu.sync_copy(zbuf.at[pl.ds(0, sz)], s_sh.at[pl.ds(zb + o, sz)])
        plsc.subcore_barrier()

        trip = jnp.maximum(0, (nch - sid + NSUB - 1) // NSUB)

        def chunk(k, _):
            st = pl.multiple_of(lo8 + (sid + k * NSUB) * CB, 8)
            pltpu.sync_copy(srcp_h.at[pl.ds(st, CB)], srcv)
            pltpu.sync_copy(dstp_h.at[pl.ds(st, CB)], dstv)
            cp = pltpu.async_copy(cin_h.at[srcv], rows, sem)
            for g in range(CB // LANES):
                pos = st + g * LANES + i16
                d = dstv[pl.ds(g * LANES, LANES)]
                inb = jnp.logical_and(pos >= lo, pos < hi)
                di = jnp.where(inb, d, N)
                idxw[0, pl.ds(g * LANES, LANES)] = di
            cp.wait()
            pltpu.sync_copy(rows, s_sh.at[idxw.at[0]], add=True)
            return 0

        lax.fori_loop(0, trip, chunk, 0)
        plsc.subcore_barrier()

        pltpu.sync_copy(
            s_sh.at[pl.ds(sid * SLICE, SLICE)],
            s_out.at[pl.ds(r * NP + sid * SLICE, SLICE)],
        )
        plsc.subcore_barrier()


def _make_agg_call(interpret=False):
    return pl.kernel(
        _agg_body,
        out_type=jax.ShapeDtypeStruct((R * NP, H), jnp.float32),
        mesh=_mesh(),
        scratch_types=[
            pltpu.VMEM((CB,), jnp.int32),
            pltpu.VMEM((CB,), jnp.int32),
            pltpu.VMEM((1, CB), jnp.int32),
            pltpu.VMEM((CB, H), jnp.float32),
            pltpu.VMEM((CB, H), jnp.float32),
            pltpu.VMEM((LANES,), jnp.int32),
            pltpu.VMEM_SHARED((NP, H), jnp.float32),
            pltpu.SemaphoreType.DMA,
        ],
        interpret=interpret,
    )


def _tc1_body(x_ref, wt_ref, bt_ref, bat_ref, did_ref, cpp_ref, ctp_ref,
              pe_ref, de_ref, sb_ref, cin0_ref, pm_ref, ct_ref):
    xw = jnp.dot(x_ref[...], wt_ref[...], preferred_element_type=jnp.float32)
    xw = xw + bt_ref[...]
    bvec = bat_ref[0, 0, :]
    ohb = (bvec[:, None] == lax.broadcasted_iota(jnp.int32, (BN, B), 1))
    dvec = did_ref[0, :]
    ohd = (dvec[:, None] == lax.broadcasted_iota(jnp.int32, (B, D), 1))
    dmat = jnp.dot(ohd.astype(jnp.float32), de_ref[...],
                   preferred_element_type=jnp.float32)
    drows = jnp.dot(ohb.astype(jnp.float32), dmat,
                    preferred_element_type=jnp.float32)
    selp = (lax.broadcasted_iota(jnp.int32, (H, P), 0)
            == LANES * lax.broadcasted_iota(jnp.int32, (H, P), 1))
    selr = (lax.broadcasted_iota(jnp.int32, (H, R), 0)
            == LANES * lax.broadcasted_iota(jnp.int32, (H, R), 1))
    cp = jnp.dot(cpp_ref[0] + cpp_ref[1], selp.astype(jnp.float32),
                 preferred_element_type=jnp.float32)
    ct_ref[...] = jnp.dot(ctp_ref[0] + ctp_ref[1], selr.astype(jnp.float32),
                          preferred_element_type=jnp.float32)
    deg = jnp.maximum(jnp.sum(cp, axis=1, keepdims=True), 1.0)
    pm = jnp.dot(cp, pe_ref[...], preferred_element_type=jnp.float32) / deg
    h = xw * drows + pm
    sb_ref[...] = h
    cin0_ref[...] = h + pm
    pm_ref[...] = pm


def _make_tc1(interpret=False):
    return pl.pallas_call(
        _tc1_body,
        grid=(GRID,),
        in_specs=[
            pl.BlockSpec((BN, TD), lambda i: (i, 0)),
            pl.BlockSpec((TD, H), lambda i: (0, 0)),
            pl.BlockSpec((1, H), lambda i: (0, 0)),
            pl.BlockSpec((1, 1, BN), lambda i: (i, 0, 0)),
            pl.BlockSpec((1, B), lambda i: (0, 0)),
            pl.BlockSpec((NSC, BN, H), lambda i: (0, i, 0)),
            pl.BlockSpec((NSC, BN, H), lambda i: (0, i, 0)),
            pl.BlockSpec((P, H), lambda i: (0, 0)),
            pl.BlockSpec((D, H), lambda i: (0, 0)),
        ],
        out_specs=[
            pl.BlockSpec((BN, H), lambda i: (i, 0)),
            pl.BlockSpec((BN, H), lambda i: (i, 0)),
            pl.BlockSpec((BN, H), lambda i: (i, 0)),
            pl.BlockSpec((BN, R), lambda i: (i, 0)),
        ],
        out_shape=[
            jax.ShapeDtypeStruct((N, H), jnp.float32),
            jax.ShapeDtypeStruct((N, H), jnp.float32),
            jax.ShapeDtypeStruct((N, H), jnp.float32),
            jax.ShapeDtypeStruct((N, R), jnp.float32),
        ],
        interpret=interpret,
    )


def _tc2_body(s_ref, ct_ref, cin_ref, pm_ref, w_ref, root_ref, b_ref,
              out_ref, cnext_ref):
    cin = cin_ref[...]
    acc = jnp.dot(cin, root_ref[...], preferred_element_type=jnp.float32)
    inv = 1.0 / jnp.maximum(ct_ref[...], 1.0)
    for r in range(R):
        mean_r = s_ref[r] * inv[:, r][:, None]
        acc = acc + jnp.dot(mean_r, w_ref[r],
                            preferred_element_type=jnp.float32)
    o = jnp.maximum(acc + b_ref[...], 0.0)
    out_ref[...] = o
    cnext_ref[...] = o + pm_ref[...]


def _make_tc2(interpret=False):
    return pl.pallas_call(
        _tc2_body,
        grid=(GRID,),
        in_specs=[
            pl.BlockSpec((R, BN, H), lambda i: (0, i, 0)),
            pl.BlockSpec((BN, R), lambda i: (i, 0)),
            pl.BlockSpec((BN, H), lambda i: (i, 0)),
            pl.BlockSpec((BN, H), lambda i: (i, 0)),
            pl.BlockSpec((R, H, H), lambda i: (0, 0, 0)),
            pl.BlockSpec((H, H), lambda i: (0, 0)),
            pl.BlockSpec((1, H), lambda i: (0, 0)),
        ],
        out_specs=[
            pl.BlockSpec((BN, H), lambda i: (i, 0)),
            pl.BlockSpec((BN, H), lambda i: (i, 0)),
        ],
        out_shape=[
            jax.ShapeDtypeStruct((N, H), jnp.float32),
            jax.ShapeDtypeStruct((N, H), jnp.float32),
        ],
        interpret=interpret,
    )


def _tc3_body(sb_ref, cu_ref, ws_ref, bs_ref, wst_ref, bst_ref,
              wg1a_ref, wg1b_ref, bg1_ref, wg2_ref, bg2_ref, out_ref):
    sv = jnp.dot(sb_ref[...], ws_ref[...],
                 preferred_element_type=jnp.float32) + bs_ref[...]
    stv = jnp.dot(cu_ref[...], wst_ref[...],
                  preferred_element_type=jnp.float32) + bst_ref[...]
    g1 = jnp.dot(sv, wg1a_ref[...], preferred_element_type=jnp.float32)
    g1 = g1 + jnp.dot(stv, wg1b_ref[...], preferred_element_type=jnp.float32)
    g1 = jnp.maximum(g1 + bg1_ref[...], 0.0)
    z = jnp.dot(g1, wg2_ref[...],
                preferred_element_type=jnp.float32) + bg2_ref[...]
    gate = 1.0 / (1.0 + jnp.exp(-z))
    out_ref[...] = gate * sv + (1.0 - gate) * stv


def _make_tc3(interpret=False):
    mat = pl.BlockSpec((H, H), lambda i: (0, 0))
    vec = pl.BlockSpec((1, H), lambda i: (0, 0))
    rows = pl.BlockSpec((BN, H), lambda i: (i, 0))
    return pl.pallas_call(
        _tc3_body,
        grid=(GRID,),
        in_specs=[rows, rows, mat, vec, mat, vec, mat, mat, vec, mat, vec],
        out_specs=rows,
        out_shape=jax.ShapeDtypeStruct((N, H), jnp.float32),
        interpret=interpret,
    )


def _run(interpret_tc, interpret_sc,
         x_text, edge_index, edge_type, batch, domain_ids, edge_property_id,
         Wt, bt, dom_emb, prop_emb, W0, root0, b0, W1, root1, b1,
         Ws, bs, Wst, bst, Wg1, bg1, Wg2, bg2):
    src = edge_index[0].astype(jnp.int32)
    dst = edge_index[1].astype(jnp.int32)
    et = edge_type.astype(jnp.int32)
    pid = jnp.clip(edge_property_id.astype(jnp.int32), 0, P - 1)

    order = jnp.argsort(et)
    ets = et[order]
    pad = jnp.zeros((CB,), jnp.int32)
    src_s = jnp.concatenate([src[order], pad])
    dst_s = jnp.concatenate([dst[order], pad])
    off = jnp.searchsorted(ets, jnp.arange(R + 1, dtype=jnp.int32)).astype(jnp.int32)
    pidorder = jnp.argsort(pid)
    srcq = jnp.concatenate([src[pidorder], pad])
    poff = jnp.searchsorted(pid[pidorder],
                            jnp.arange(P + 1, dtype=jnp.int32)).astype(jnp.int32)
    off16 = jnp.concatenate(
        [off, poff, jnp.zeros((LANES - R - P - 2,), jnp.int32)])
    zrs = jnp.zeros((CB, H), jnp.float32)

    cnt_call = _make_cnt_call(interpret_sc)
    agg_call = _make_agg_call(interpret_sc)
    tc1 = _make_tc1(interpret_tc)
    tc2 = _make_tc2(interpret_tc)
    tc3 = _make_tc3(interpret_tc)

    outc, outp = cnt_call(dst_s, srcq, off16, zrs)
    ctp = outc.reshape(NSC, NP, H)
    cpp = outp.reshape(NSC, NP, H)

    bat3 = batch.astype(jnp.int32).reshape(GRID, 1, BN)
    did2 = domain_ids.astype(jnp.int32).reshape(1, B)
    sb, cin0, pm, ct = tc1(x_text, Wt, bt.reshape(1, H), bat3, did2,
                           cpp, ctp, prop_emb, dom_emb)

    s0 = agg_call(cin0, src_s, dst_s, off16, zrs)
    relu0, cin1 = tc2(s0.reshape(R, NP, H), ct, cin0, pm,
                      W0, root0, b0.reshape(1, H))

    s1 = agg_call(cin1, src_s, dst_s, off16, zrs)
    relu1, _ = tc2(s1.reshape(R, NP, H), ct, cin1, pm,
                   W1, root1, b1.reshape(1, H))

    return tc3(sb, relu1, Ws, bs.reshape(1, H), Wst, bst.reshape(1, H),
               Wg1[:H], Wg1[H:], bg1.reshape(1, H), Wg2, bg2.reshape(1, H))


def kernel(x_text, edge_index, edge_type, batch, domain_ids, edge_property_id,
           Wt, bt, dom_emb, prop_emb, W0, root0, b0, W1, root1, b1,
           Ws, bs, Wst, bst, Wg1, bg1, Wg2, bg2):
    return _run(False, False,
                x_text, edge_index, edge_type, batch, domain_ids,
                edge_property_id, Wt, bt, dom_emb, prop_emb,
                W0, root0, b0, W1, root1, b1,
                Ws, bs, Wst, bst, Wg1, bg1, Wg2, bg2)


# speedup vs baseline: 9.5146x; 1.0846x over previous
import functools

import jax
import jax.numpy as jnp
from jax import lax
from jax.experimental import pallas as pl
from jax.experimental.pallas import tpu as pltpu
from jax.experimental.pallas import tpu_sc as plsc

N = 10000
E = 320000
TD = 256
H = 128
R = 8
D = 16
P = 4
B = 8

NSC = 2
NSUB = 16
LANES = 16

CB = 128
NP = 10112
SLICE = NP // NSUB
NPP = 40064
NR = 81920
EC = E // (NSC * NSUB)
PCB = 80

BN = 400
GRID = N // BN

def _mesh():
    return plsc.VectorSubcoreMesh(
        core_axis_name="c", subcore_axis_name="s",
        num_cores=NSC, num_subcores=NSUB,
    )


def _cnt_body(dstp_h, srcq_h, off_h, zrs_h, outc_h, outp_h,
              dstv, idxw, rows, zbuf, offv, s_sh):
    cid = lax.axis_index("c")
    sid = lax.axis_index("s")
    i16 = lax.broadcasted_iota(jnp.int32, (LANES,), 0)

    pltpu.sync_copy(off_h, offv)
    pltpu.sync_copy(zrs_h, zbuf)
    offvec = offv[...]

    def zero_slice():
        zb = sid * SLICE
        for o, sz in ((0, CB), (CB, CB), (2 * CB, CB), (3 * CB, CB),
                      (4 * CB, SLICE - 4 * CB)):
            pltpu.sync_copy(zbuf.at[pl.ds(0, sz)], s_sh.at[pl.ds(zb + o, sz)])

    def fill_onehot(col):
        def rb(i, _):
            for cg in range(H // LANES):
                v = jnp.where(cg * LANES + i16 == col, 1.0, 0.0)
                rows[i, pl.ds(cg * LANES, LANES)] = v
            return 0
        lax.fori_loop(0, CB, rb, 0)

    def add_range(idx_h, lo, hi):
        lo8 = jnp.bitwise_and(lo, -8)
        nch = (hi - lo8 + CB - 1) // CB
        trip = jnp.maximum(0, (nch - sid + NSUB - 1) // NSUB)

        def chunk(k, _):
            st = pl.multiple_of(lo8 + (sid + k * NSUB) * CB, 8)
            pltpu.sync_copy(idx_h.at[pl.ds(st, CB)], dstv)
            for g in range(CB // LANES):
                pos = st + g * LANES + i16
                d = dstv[pl.ds(g * LANES, LANES)]
                inb = jnp.logical_and(pos >= lo, pos < hi)
                idxw[0, pl.ds(g * LANES, LANES)] = jnp.where(inb, d, N)
            pltpu.sync_copy(rows, s_sh.at[idxw.at[0]], add=True)
            return 0

        lax.fori_loop(0, trip, chunk, 0)

    half = R // NSC
    zero_slice()
    plsc.subcore_barrier()
    for j in range(half):
        r = cid * half + j
        lo = jnp.where(cid == 0, offvec[j], offvec[half + j])
        hi = jnp.where(cid == 0, offvec[j + 1], offvec[half + j + 1])
        fill_onehot(r * LANES)
        add_range(dstp_h, lo, hi)
    plsc.subcore_barrier()
    pltpu.sync_copy(s_sh.at[pl.ds(sid * SLICE, SLICE)],
                    outc_h.at[pl.ds(cid * NP + sid * SLICE, SLICE)])
    plsc.subcore_barrier()

    phalf = P // NSC
    zero_slice()
    plsc.subcore_barrier()
    for j in range(phalf):
        p = cid * phalf + j
        lo = jnp.where(cid == 0, offvec[R + 1 + j], offvec[R + 1 + phalf + j])
        hi = jnp.where(cid == 0, offvec[R + 2 + j], offvec[R + 2 + phalf + j])
        fill_onehot(p * LANES)
        add_range(srcq_h, lo, hi)
    plsc.subcore_barrier()
    pltpu.sync_copy(s_sh.at[pl.ds(sid * SLICE, SLICE)],
                    outp_h.at[pl.ds(cid * NP + sid * SLICE, SLICE)])


def _make_cnt_call(interpret=False):
    return pl.kernel(
        _cnt_body,
        out_type=(
            jax.ShapeDtypeStruct((NSC * NP, H), jnp.float32),
            jax.ShapeDtypeStruct((NSC * NP, H), jnp.float32),
        ),
        mesh=_mesh(),
        scratch_types=[
            pltpu.VMEM((CB,), jnp.int32),
            pltpu.VMEM((1, CB), jnp.int32),
            pltpu.VMEM((CB, H), jnp.float32),
            pltpu.VMEM((CB, H), jnp.float32),
            pltpu.VMEM((LANES,), jnp.int32),
            pltpu.VMEM_SHARED((NP, H), jnp.float32),
        ],
        interpret=interpret,
    )


def _agg_body(cin_h, srcp_h, dstp_h, off_h, zrs_h,
              s_out,
              srcv, dstv, srcv2, dstv2, idxw, rows, rows2, zbuf, offv,
              s_sh, sem):
    cid = lax.axis_index("c")
    sid = lax.axis_index("s")
    i16 = lax.broadcasted_iota(jnp.int32, (LANES,), 0)

    pltpu.sync_copy(off_h, offv)
    pltpu.sync_copy(zrs_h.at[pl.ds(0, 64)], zbuf)
    offvec = offv[...]

    bufs = ((srcv, dstv, rows), (srcv2, dstv2, rows2))
    half = R // NSC
    for j in range(half):
        r = cid * half + j
        lo = jnp.where(cid == 0, offvec[j], offvec[half + j])
        hi = jnp.where(cid == 0, offvec[j + 1], offvec[half + j + 1])
        lo8 = jnp.bitwise_and(lo, -8)
        nch = (hi - lo8 + CB - 1) // CB

        zb = sid * SLICE
        o = 0
        while o < SLICE:
            sz = min(64, SLICE - o)
            pltpu.sync_copy(zbuf.at[pl.ds(0, sz)], s_sh.at[pl.ds(zb + o, sz)])
            o += sz
        plsc.subcore_barrier()

        trip = jnp.maximum(0, (nch - sid + NSUB - 1) // NSUB)

        def outer(i, _):
            k0 = i * 2
            for b, (sv, dv, rb) in enumerate(bufs):
                @pl.when(k0 + b < trip)
                def _(b=b, sv=sv, dv=dv, rb=rb):
                    st = pl.multiple_of(
                        lo8 + (sid + (k0 + b) * NSUB) * CB, 8)
                    pltpu.sync_copy(srcp_h.at[pl.ds(st, CB)], sv)
                    pltpu.sync_copy(dstp_h.at[pl.ds(st, CB)], dv)
                    pltpu.async_copy(cin_h.at[sv], rb, sem)
            for b, (sv, dv, rb) in enumerate(bufs):
                @pl.when(k0 + b < trip)
                def _(b=b, sv=sv, dv=dv, rb=rb):
                    st = pl.multiple_of(
                        lo8 + (sid + (k0 + b) * NSUB) * CB, 8)
                    for g in range(CB // LANES):
                        pos = st + g * LANES + i16
                        d = dv[pl.ds(g * LANES, LANES)]
                        inb = jnp.logical_and(pos >= lo, pos < hi)
                        idxw[0, pl.ds(g * LANES, LANES)] = jnp.where(inb, d, N)
                    pltpu.make_async_copy(cin_h.at[sv], rb, sem).wait()
                    pltpu.sync_copy(rb, s_sh.at[idxw.at[0]], add=True)
            return 0

        lax.fori_loop(0, (trip + 1) // 2, outer, 0)
        plsc.subcore_barrier()

        pltpu.sync_copy(
            s_sh.at[pl.ds(sid * SLICE, SLICE)],
            s_out.at[pl.ds(r * NP + sid * SLICE, SLICE)],
        )
        plsc.subcore_barrier()


def _make_agg_call(interpret=False):
    return pl.kernel(
        _agg_body,
        out_type=jax.ShapeDtypeStruct((R * NP, H), jnp.float32),
        mesh=_mesh(),
        scratch_types=[
            pltpu.VMEM((CB,), jnp.int32),
            pltpu.VMEM((CB,), jnp.int32),
            pltpu.VMEM((CB,), jnp.int32),
            pltpu.VMEM((CB,), jnp.int32),
            pltpu.VMEM((1, CB), jnp.int32),
            pltpu.VMEM((CB, H), jnp.float32),
            pltpu.VMEM((CB, H), jnp.float32),
            pltpu.VMEM((64, H), jnp.float32),
            pltpu.VMEM((LANES,), jnp.int32),
            pltpu.VMEM_SHARED((NP, H), jnp.float32),
            pltpu.SemaphoreType.DMA,
        ],
        interpret=interpret,
    )


def _tc1_body(x_ref, wt_ref, bt_ref, bat_ref, did_ref, cpp_ref, ctp_ref,
              pe_ref, de_ref, sb_ref, cin0_ref, pm_ref, ct_ref):
    xw = jnp.dot(x_ref[...], wt_ref[...], preferred_element_type=jnp.float32)
    xw = xw + bt_ref[...]
    bvec = bat_ref[0, 0, :]
    ohb = (bvec[:, None] == lax.broadcasted_iota(jnp.int32, (BN, B), 1))
    dvec = did_ref[0, :]
    ohd = (dvec[:, None] == lax.broadcasted_iota(jnp.int32, (B, D), 1))
    dmat = jnp.dot(ohd.astype(jnp.float32), de_ref[...],
                   preferred_element_type=jnp.float32)
    drows = jnp.dot(ohb.astype(jnp.float32), dmat,
                    preferred_element_type=jnp.float32)
    selp = (lax.broadcasted_iota(jnp.int32, (H, P), 0)
            == LANES * lax.broadcasted_iota(jnp.int32, (H, P), 1))
    selr = (lax.broadcasted_iota(jnp.int32, (H, R), 0)
            == LANES * lax.broadcasted_iota(jnp.int32, (H, R), 1))
    cp = jnp.dot(cpp_ref[0] + cpp_ref[1], selp.astype(jnp.float32),
                 preferred_element_type=jnp.float32)
    ct_ref[...] = jnp.dot(ctp_ref[0] + ctp_ref[1], selr.astype(jnp.float32),
                          preferred_element_type=jnp.float32)
    deg = jnp.maximum(jnp.sum(cp, axis=1, keepdims=True), 1.0)
    pm = jnp.dot(cp, pe_ref[...], preferred_element_type=jnp.float32) / deg
    h = xw * drows + pm
    sb_ref[...] = h
    cin0_ref[...] = h + pm
    pm_ref[...] = pm


def _make_tc1(interpret=False):
    return pl.pallas_call(
        _tc1_body,
        grid=(GRID,),
        in_specs=[
            pl.BlockSpec((BN, TD), lambda i: (i, 0)),
            pl.BlockSpec((TD, H), lambda i: (0, 0)),
            pl.BlockSpec((1, H), lambda i: (0, 0)),
            pl.BlockSpec((1, 1, BN), lambda i: (i, 0, 0)),
            pl.BlockSpec((1, B), lambda i: (0, 0)),
            pl.BlockSpec((NSC, BN, H), lambda i: (0, i, 0)),
            pl.BlockSpec((NSC, BN, H), lambda i: (0, i, 0)),
            pl.BlockSpec((P, H), lambda i: (0, 0)),
            pl.BlockSpec((D, H), lambda i: (0, 0)),
        ],
        out_specs=[
            pl.BlockSpec((BN, H), lambda i: (i, 0)),
            pl.BlockSpec((BN, H), lambda i: (i, 0)),
            pl.BlockSpec((BN, H), lambda i: (i, 0)),
            pl.BlockSpec((BN, R), lambda i: (i, 0)),
        ],
        out_shape=[
            jax.ShapeDtypeStruct((N, H), jnp.float32),
            jax.ShapeDtypeStruct((N, H), jnp.float32),
            jax.ShapeDtypeStruct((N, H), jnp.float32),
            jax.ShapeDtypeStruct((N, R), jnp.float32),
        ],
        interpret=interpret,
    )


def _tc2_body(s_ref, ct_ref, cin_ref, pm_ref, w_ref, root_ref, b_ref,
              out_ref, cnext_ref):
    cin = cin_ref[...]
    acc = jnp.dot(cin, root_ref[...], preferred_element_type=jnp.float32)
    inv = 1.0 / jnp.maximum(ct_ref[...], 1.0)
    for r in range(R):
        mean_r = s_ref[r] * inv[:, r][:, None]
        acc = acc + jnp.dot(mean_r, w_ref[r],
                            preferred_element_type=jnp.float32)
    o = jnp.maximum(acc + b_ref[...], 0.0)
    out_ref[...] = o
    cnext_ref[...] = o + pm_ref[...]


def _make_tc2(interpret=False):
    return pl.pallas_call(
        _tc2_body,
        grid=(GRID,),
        in_specs=[
            pl.BlockSpec((R, BN, H), lambda i: (0, i, 0)),
            pl.BlockSpec((BN, R), lambda i: (i, 0)),
            pl.BlockSpec((BN, H), lambda i: (i, 0)),
            pl.BlockSpec((BN, H), lambda i: (i, 0)),
            pl.BlockSpec((R, H, H), lambda i: (0, 0, 0)),
            pl.BlockSpec((H, H), lambda i: (0, 0)),
            pl.BlockSpec((1, H), lambda i: (0, 0)),
        ],
        out_specs=[
            pl.BlockSpec((BN, H), lambda i: (i, 0)),
            pl.BlockSpec((BN, H), lambda i: (i, 0)),
        ],
        out_shape=[
            jax.ShapeDtypeStruct((N, H), jnp.float32),
            jax.ShapeDtypeStruct((N, H), jnp.float32),
        ],
        interpret=interpret,
    )


def _tc3_body(sb_ref, cu_ref, ws_ref, bs_ref, wst_ref, bst_ref,
              wg1a_ref, wg1b_ref, bg1_ref, wg2_ref, bg2_ref, out_ref):
    sv = jnp.dot(sb_ref[...], ws_ref[...],
                 preferred_element_type=jnp.float32) + bs_ref[...]
    stv = jnp.dot(cu_ref[...], wst_ref[...],
                  preferred_element_type=jnp.float32) + bst_ref[...]
    g1 = jnp.dot(sv, wg1a_ref[...], preferred_element_type=jnp.float32)
    g1 = g1 + jnp.dot(stv, wg1b_ref[...], preferred_element_type=jnp.float32)
    g1 = jnp.maximum(g1 + bg1_ref[...], 0.0)
    z = jnp.dot(g1, wg2_ref[...],
                preferred_element_type=jnp.float32) + bg2_ref[...]
    gate = 1.0 / (1.0 + jnp.exp(-z))
    out_ref[...] = gate * sv + (1.0 - gate) * stv


def _make_tc3(interpret=False):
    mat = pl.BlockSpec((H, H), lambda i: (0, 0))
    vec = pl.BlockSpec((1, H), lambda i: (0, 0))
    rows = pl.BlockSpec((BN, H), lambda i: (i, 0))
    return pl.pallas_call(
        _tc3_body,
        grid=(GRID,),
        in_specs=[rows, rows, mat, vec, mat, vec, mat, mat, vec, mat, vec],
        out_specs=rows,
        out_shape=jax.ShapeDtypeStruct((N, H), jnp.float32),
        interpret=interpret,
    )


def _run(interpret_tc, interpret_sc,
         x_text, edge_index, edge_type, batch, domain_ids, edge_property_id,
         Wt, bt, dom_emb, prop_emb, W0, root0, b0, W1, root1, b1,
         Ws, bs, Wst, bst, Wg1, bg1, Wg2, bg2):
    src = edge_index[0].astype(jnp.int32)
    dst = edge_index[1].astype(jnp.int32)
    et = edge_type.astype(jnp.int32)
    pid = jnp.clip(edge_property_id.astype(jnp.int32), 0, P - 1)

    order = jnp.argsort(et)
    ets = et[order]
    pad = jnp.zeros((CB,), jnp.int32)
    src_s = jnp.concatenate([src[order], pad])
    dst_s = jnp.concatenate([dst[order], pad])
    off = jnp.searchsorted(ets, jnp.arange(R + 1, dtype=jnp.int32)).astype(jnp.int32)
    pidorder = jnp.argsort(pid)
    srcq = jnp.concatenate([src[pidorder], pad])
    poff = jnp.searchsorted(pid[pidorder],
                            jnp.arange(P + 1, dtype=jnp.int32)).astype(jnp.int32)
    off16 = jnp.concatenate(
        [off, poff, jnp.zeros((LANES - R - P - 2,), jnp.int32)])
    zrs = jnp.zeros((CB, H), jnp.float32)

    cnt_call = _make_cnt_call(interpret_sc)
    agg_call = _make_agg_call(interpret_sc)
    tc1 = _make_tc1(interpret_tc)
    tc2 = _make_tc2(interpret_tc)
    tc3 = _make_tc3(interpret_tc)

    outc, outp = cnt_call(dst_s, srcq, off16, zrs)
    ctp = outc.reshape(NSC, NP, H)
    cpp = outp.reshape(NSC, NP, H)

    bat3 = batch.astype(jnp.int32).reshape(GRID, 1, BN)
    did2 = domain_ids.astype(jnp.int32).reshape(1, B)
    sb, cin0, pm, ct = tc1(x_text, Wt, bt.reshape(1, H), bat3, did2,
                           cpp, ctp, prop_emb, dom_emb)

    s0 = agg_call(cin0, src_s, dst_s, off16, zrs)
    relu0, cin1 = tc2(s0.reshape(R, NP, H), ct, cin0, pm,
                      W0, root0, b0.reshape(1, H))

    s1 = agg_call(cin1, src_s, dst_s, off16, zrs)
    relu1, _ = tc2(s1.reshape(R, NP, H), ct, cin1, pm,
                   W1, root1, b1.reshape(1, H))

    return tc3(sb, relu1, Ws, bs.reshape(1, H), Wst, bst.reshape(1, H),
               Wg1[:H], Wg1[H:], bg1.reshape(1, H), Wg2, bg2.reshape(1, H))


def kernel(x_text, edge_index, edge_type, batch, domain_ids, edge_property_id,
           Wt, bt, dom_emb, prop_emb, W0, root0, b0, W1, root1, b1,
           Ws, bs, Wst, bst, Wg1, bg1, Wg2, bg2):
    return _run(False, False,
                x_text, edge_index, edge_type, batch, domain_ids,
                edge_property_id, Wt, bt, dom_emb, prop_emb,
                W0, root0, b0, W1, root1, b1,
                Ws, bs, Wst, bst, Wg1, bg1, Wg2, bg2)


# speedup vs baseline: 9.5182x; 1.0004x over previous
import functools

import jax
import jax.numpy as jnp
from jax import lax
from jax.experimental import pallas as pl
from jax.experimental.pallas import tpu as pltpu
from jax.experimental.pallas import tpu_sc as plsc

N = 10000
E = 320000
TD = 256
H = 128
R = 8
D = 16
P = 4
B = 8

NSC = 2
NSUB = 16
LANES = 16

CB = 128
NP = 10112
SLICE = NP // NSUB
NPP = 40064
NR = 81920
EC = E // (NSC * NSUB)
PCB = 80

BN = 400
GRID = N // BN

def _mesh():
    return plsc.VectorSubcoreMesh(
        core_axis_name="c", subcore_axis_name="s",
        num_cores=NSC, num_subcores=NSUB,
    )


def _cnt_body(dstp_h, srcq_h, off_h, zrs_h, outc_h, outp_h,
              dstv, idxw, rows, zbuf, offv, s_sh):
    cid = lax.axis_index("c")
    sid = lax.axis_index("s")
    i16 = lax.broadcasted_iota(jnp.int32, (LANES,), 0)

    pltpu.sync_copy(off_h, offv)
    pltpu.sync_copy(zrs_h, zbuf)
    offvec = offv[...]

    def zero_slice():
        zb = sid * SLICE
        for o, sz in ((0, CB), (CB, CB), (2 * CB, CB), (3 * CB, CB),
                      (4 * CB, SLICE - 4 * CB)):
            pltpu.sync_copy(zbuf.at[pl.ds(0, sz)], s_sh.at[pl.ds(zb + o, sz)])

    def fill_onehot(col):
        def rb(i, _):
            for cg in range(H // LANES):
                v = jnp.where(cg * LANES + i16 == col, 1.0, 0.0)
                rows[i, pl.ds(cg * LANES, LANES)] = v
            return 0
        lax.fori_loop(0, CB, rb, 0)

    def add_range(idx_h, lo, hi):
        lo8 = jnp.bitwise_and(lo, -8)
        nch = (hi - lo8 + CB - 1) // CB
        trip = jnp.maximum(0, (nch - sid + NSUB - 1) // NSUB)

        def chunk(k, _):
            st = pl.multiple_of(lo8 + (sid + k * NSUB) * CB, 8)
            pltpu.sync_copy(idx_h.at[pl.ds(st, CB)], dstv)
            for g in range(CB // LANES):
                pos = st + g * LANES + i16
                d = dstv[pl.ds(g * LANES, LANES)]
                inb = jnp.logical_and(pos >= lo, pos < hi)
                idxw[0, pl.ds(g * LANES, LANES)] = jnp.where(inb, d, N)
            pltpu.sync_copy(rows, s_sh.at[idxw.at[0]], add=True)
            return 0

        lax.fori_loop(0, trip, chunk, 0)

    half = R // NSC
    zero_slice()
    plsc.subcore_barrier()
    for j in range(half):
        r = cid * half + j
        lo = jnp.where(cid == 0, offvec[j], offvec[half + j])
        hi = jnp.where(cid == 0, offvec[j + 1], offvec[half + j + 1])
        fill_onehot(r * LANES)
        add_range(dstp_h, lo, hi)
    plsc.subcore_barrier()
    pltpu.sync_copy(s_sh.at[pl.ds(sid * SLICE, SLICE)],
                    outc_h.at[pl.ds(cid * NP + sid * SLICE, SLICE)])
    plsc.subcore_barrier()

    phalf = P // NSC
    zero_slice()
    plsc.subcore_barrier()
    for j in range(phalf):
        p = cid * phalf + j
        lo = jnp.where(cid == 0, offvec[R + 1 + j], offvec[R + 1 + phalf + j])
        hi = jnp.where(cid == 0, offvec[R + 2 + j], offvec[R + 2 + phalf + j])
        fill_onehot(p * LANES)
        add_range(srcq_h, lo, hi)
    plsc.subcore_barrier()
    pltpu.sync_copy(s_sh.at[pl.ds(sid * SLICE, SLICE)],
                    outp_h.at[pl.ds(cid * NP + sid * SLICE, SLICE)])


def _make_cnt_call(interpret=False):
    return pl.kernel(
        _cnt_body,
        out_type=(
            jax.ShapeDtypeStruct((NSC * NP, H), jnp.float32),
            jax.ShapeDtypeStruct((NSC * NP, H), jnp.float32),
        ),
        mesh=_mesh(),
        scratch_types=[
            pltpu.VMEM((CB,), jnp.int32),
            pltpu.VMEM((1, CB), jnp.int32),
            pltpu.VMEM((CB, H), jnp.float32),
            pltpu.VMEM((CB, H), jnp.float32),
            pltpu.VMEM((LANES,), jnp.int32),
            pltpu.VMEM_SHARED((NP, H), jnp.float32),
        ],
        interpret=interpret,
    )


def _agg_body(cin_h, srcp_h, dstp_h, off_h, zrs_h,
              s_out,
              srcv, dstv, srcv2, dstv2, idxw, idxw2, rows, rows2, zbuf, offv,
              s_sh, sem, sem2):
    cid = lax.axis_index("c")
    sid = lax.axis_index("s")
    i16 = lax.broadcasted_iota(jnp.int32, (LANES,), 0)

    pltpu.sync_copy(off_h, offv)
    pltpu.sync_copy(zrs_h.at[pl.ds(0, 64)], zbuf)
    offvec = offv[...]

    bufs = ((srcv, dstv, rows), (srcv2, dstv2, rows2))
    half = R // NSC
    for j in range(half):
        r = cid * half + j
        lo = jnp.where(cid == 0, offvec[j], offvec[half + j])
        hi = jnp.where(cid == 0, offvec[j + 1], offvec[half + j + 1])
        lo8 = jnp.bitwise_and(lo, -8)
        nch = (hi - lo8 + CB - 1) // CB

        zb = sid * SLICE
        o = 0
        while o < SLICE:
            sz = min(64, SLICE - o)
            pltpu.sync_copy(zbuf.at[pl.ds(0, sz)], s_sh.at[pl.ds(zb + o, sz)])
            o += sz
        plsc.subcore_barrier()

        trip = jnp.maximum(0, (nch - sid + NSUB - 1) // NSUB)

        def outer(i, _):
            k0 = i * 2
            for b, (sv, dv, rb) in enumerate(bufs):
                @pl.when(k0 + b < trip)
                def _(b=b, sv=sv, dv=dv, rb=rb):
                    st = pl.multiple_of(
                        lo8 + (sid + (k0 + b) * NSUB) * CB, 8)
                    pltpu.sync_copy(srcp_h.at[pl.ds(st, CB)], sv)
                    pltpu.sync_copy(dstp_h.at[pl.ds(st, CB)], dv)
                    pltpu.async_copy(cin_h.at[sv], rb, sem)
            for b, (sv, dv, rb) in enumerate(bufs):
                @pl.when(k0 + b < trip)
                def _(b=b, sv=sv, dv=dv, rb=rb):
                    iw = idxw if b == 0 else idxw2
                    st = pl.multiple_of(
                        lo8 + (sid + (k0 + b) * NSUB) * CB, 8)
                    for g in range(CB // LANES):
                        pos = st + g * LANES + i16
                        d = dv[pl.ds(g * LANES, LANES)]
                        inb = jnp.logical_and(pos >= lo, pos < hi)
                        iw[0, pl.ds(g * LANES, LANES)] = jnp.where(inb, d, N)
                    pltpu.make_async_copy(cin_h.at[sv], rb, sem).wait()
                    pltpu.async_copy(rb, s_sh.at[iw.at[0]], sem2, add=True)
            for b, (sv, dv, rb) in enumerate(bufs):
                @pl.when(k0 + b < trip)
                def _(b=b, sv=sv, dv=dv, rb=rb):
                    iw = idxw if b == 0 else idxw2
                    pltpu.make_async_copy(
                        rb, s_sh.at[iw.at[0]], sem2).wait()
            return 0

        lax.fori_loop(0, (trip + 1) // 2, outer, 0)
        plsc.subcore_barrier()

        pltpu.sync_copy(
            s_sh.at[pl.ds(sid * SLICE, SLICE)],
            s_out.at[pl.ds(r * NP + sid * SLICE, SLICE)],
        )
        plsc.subcore_barrier()


def _make_agg_call(interpret=False):
    return pl.kernel(
        _agg_body,
        out_type=jax.ShapeDtypeStruct((R * NP, H), jnp.float32),
        mesh=_mesh(),
        scratch_types=[
            pltpu.VMEM((CB,), jnp.int32),
            pltpu.VMEM((CB,), jnp.int32),
            pltpu.VMEM((CB,), jnp.int32),
            pltpu.VMEM((CB,), jnp.int32),
            pltpu.VMEM((1, CB), jnp.int32),
            pltpu.VMEM((1, CB), jnp.int32),
            pltpu.VMEM((CB, H), jnp.float32),
            pltpu.VMEM((CB, H), jnp.float32),
            pltpu.VMEM((64, H), jnp.float32),
            pltpu.VMEM((LANES,), jnp.int32),
            pltpu.VMEM_SHARED((NP, H), jnp.float32),
            pltpu.SemaphoreType.DMA,
            pltpu.SemaphoreType.DMA,
        ],
        interpret=interpret,
    )


def _tc1_body(x_ref, wt_ref, bt_ref, bat_ref, did_ref, cpp_ref, ctp_ref,
              pe_ref, de_ref, sb_ref, cin0_ref, pm_ref, ct_ref):
    xw = jnp.dot(x_ref[...], wt_ref[...], preferred_element_type=jnp.float32)
    xw = xw + bt_ref[...]
    bvec = bat_ref[0, 0, :]
    ohb = (bvec[:, None] == lax.broadcasted_iota(jnp.int32, (BN, B), 1))
    dvec = did_ref[0, :]
    ohd = (dvec[:, None] == lax.broadcasted_iota(jnp.int32, (B, D), 1))
    dmat = jnp.dot(ohd.astype(jnp.float32), de_ref[...],
                   preferred_element_type=jnp.float32)
    drows = jnp.dot(ohb.astype(jnp.float32), dmat,
                    preferred_element_type=jnp.float32)
    selp = (lax.broadcasted_iota(jnp.int32, (H, P), 0)
            == LANES * lax.broadcasted_iota(jnp.int32, (H, P), 1))
    selr = (lax.broadcasted_iota(jnp.int32, (H, R), 0)
            == LANES * lax.broadcasted_iota(jnp.int32, (H, R), 1))
    cp = jnp.dot(cpp_ref[0] + cpp_ref[1], selp.astype(jnp.float32),
                 preferred_element_type=jnp.float32)
    ct_ref[...] = jnp.dot(ctp_ref[0] + ctp_ref[1], selr.astype(jnp.float32),
                          preferred_element_type=jnp.float32)
    deg = jnp.maximum(jnp.sum(cp, axis=1, keepdims=True), 1.0)
    pm = jnp.dot(cp, pe_ref[...], preferred_element_type=jnp.float32) / deg
    h = xw * drows + pm
    sb_ref[...] = h
    cin0_ref[...] = h + pm
    pm_ref[...] = pm


def _make_tc1(interpret=False):
    return pl.pallas_call(
        _tc1_body,
        grid=(GRID,),
        in_specs=[
            pl.BlockSpec((BN, TD), lambda i: (i, 0)),
            pl.BlockSpec((TD, H), lambda i: (0, 0)),
            pl.BlockSpec((1, H), lambda i: (0, 0)),
            pl.BlockSpec((1, 1, BN), lambda i: (i, 0, 0)),
            pl.BlockSpec((1, B), lambda i: (0, 0)),
            pl.BlockSpec((NSC, BN, H), lambda i: (0, i, 0)),
            pl.BlockSpec((NSC, BN, H), lambda i: (0, i, 0)),
            pl.BlockSpec((P, H), lambda i: (0, 0)),
            pl.BlockSpec((D, H), lambda i: (0, 0)),
        ],
        out_specs=[
            pl.BlockSpec((BN, H), lambda i: (i, 0)),
            pl.BlockSpec((BN, H), lambda i: (i, 0)),
            pl.BlockSpec((BN, H), lambda i: (i, 0)),
            pl.BlockSpec((BN, R), lambda i: (i, 0)),
        ],
        out_shape=[
            jax.ShapeDtypeStruct((N, H), jnp.float32),
            jax.ShapeDtypeStruct((N, H), jnp.float32),
            jax.ShapeDtypeStruct((N, H), jnp.float32),
            jax.ShapeDtypeStruct((N, R), jnp.float32),
        ],
        interpret=interpret,
    )


def _tc2_body(s_ref, ct_ref, cin_ref, pm_ref, w_ref, root_ref, b_ref,
              out_ref, cnext_ref):
    cin = cin_ref[...]
    acc = jnp.dot(cin, root_ref[...], preferred_element_type=jnp.float32)
    inv = 1.0 / jnp.maximum(ct_ref[...], 1.0)
    for r in range(R):
        mean_r = s_ref[r] * inv[:, r][:, None]
        acc = acc + jnp.dot(mean_r, w_ref[r],
                            preferred_element_type=jnp.float32)
    o = jnp.maximum(acc + b_ref[...], 0.0)
    out_ref[...] = o
    cnext_ref[...] = o + pm_ref[...]


def _make_tc2(interpret=False):
    return pl.pallas_call(
        _tc2_body,
        grid=(GRID,),
        in_specs=[
            pl.BlockSpec((R, BN, H), lambda i: (0, i, 0)),
            pl.BlockSpec((BN, R), lambda i: (i, 0)),
            pl.BlockSpec((BN, H), lambda i: (i, 0)),
            pl.BlockSpec((BN, H), lambda i: (i, 0)),
            pl.BlockSpec((R, H, H), lambda i: (0, 0, 0)),
            pl.BlockSpec((H, H), lambda i: (0, 0)),
            pl.BlockSpec((1, H), lambda i: (0, 0)),
        ],
        out_specs=[
            pl.BlockSpec((BN, H), lambda i: (i, 0)),
            pl.BlockSpec((BN, H), lambda i: (i, 0)),
        ],
        out_shape=[
            jax.ShapeDtypeStruct((N, H), jnp.float32),
            jax.ShapeDtypeStruct((N, H), jnp.float32),
        ],
        interpret=interpret,
    )


def _tc3_body(sb_ref, cu_ref, ws_ref, bs_ref, wst_ref, bst_ref,
              wg1a_ref, wg1b_ref, bg1_ref, wg2_ref, bg2_ref, out_ref):
    sv = jnp.dot(sb_ref[...], ws_ref[...],
                 preferred_element_type=jnp.float32) + bs_ref[...]
    stv = jnp.dot(cu_ref[...], wst_ref[...],
                  preferred_element_type=jnp.float32) + bst_ref[...]
    g1 = jnp.dot(sv, wg1a_ref[...], preferred_element_type=jnp.float32)
    g1 = g1 + jnp.dot(stv, wg1b_ref[...], preferred_element_type=jnp.float32)
    g1 = jnp.maximum(g1 + bg1_ref[...], 0.0)
    z = jnp.dot(g1, wg2_ref[...],
                preferred_element_type=jnp.float32) + bg2_ref[...]
    gate = 1.0 / (1.0 + jnp.exp(-z))
    out_ref[...] = gate * sv + (1.0 - gate) * stv


def _make_tc3(interpret=False):
    mat = pl.BlockSpec((H, H), lambda i: (0, 0))
    vec = pl.BlockSpec((1, H), lambda i: (0, 0))
    rows = pl.BlockSpec((BN, H), lambda i: (i, 0))
    return pl.pallas_call(
        _tc3_body,
        grid=(GRID,),
        in_specs=[rows, rows, mat, vec, mat, vec, mat, mat, vec, mat, vec],
        out_specs=rows,
        out_shape=jax.ShapeDtypeStruct((N, H), jnp.float32),
        interpret=interpret,
    )


def _run(interpret_tc, interpret_sc,
         x_text, edge_index, edge_type, batch, domain_ids, edge_property_id,
         Wt, bt, dom_emb, prop_emb, W0, root0, b0, W1, root1, b1,
         Ws, bs, Wst, bst, Wg1, bg1, Wg2, bg2):
    src = edge_index[0].astype(jnp.int32)
    dst = edge_index[1].astype(jnp.int32)
    et = edge_type.astype(jnp.int32)
    pid = jnp.clip(edge_property_id.astype(jnp.int32), 0, P - 1)

    order = jnp.argsort(et)
    ets = et[order]
    pad = jnp.zeros((CB,), jnp.int32)
    src_s = jnp.concatenate([src[order], pad])
    dst_s = jnp.concatenate([dst[order], pad])
    off = jnp.searchsorted(ets, jnp.arange(R + 1, dtype=jnp.int32)).astype(jnp.int32)
    pidorder = jnp.argsort(pid)
    srcq = jnp.concatenate([src[pidorder], pad])
    poff = jnp.searchsorted(pid[pidorder],
                            jnp.arange(P + 1, dtype=jnp.int32)).astype(jnp.int32)
    off16 = jnp.concatenate(
        [off, poff, jnp.zeros((LANES - R - P - 2,), jnp.int32)])
    zrs = jnp.zeros((CB, H), jnp.float32)

    cnt_call = _make_cnt_call(interpret_sc)
    agg_call = _make_agg_call(interpret_sc)
    tc1 = _make_tc1(interpret_tc)
    tc2 = _make_tc2(interpret_tc)
    tc3 = _make_tc3(interpret_tc)

    outc, outp = cnt_call(dst_s, srcq, off16, zrs)
    ctp = outc.reshape(NSC, NP, H)
    cpp = outp.reshape(NSC, NP, H)

    bat3 = batch.astype(jnp.int32).reshape(GRID, 1, BN)
    did2 = domain_ids.astype(jnp.int32).reshape(1, B)
    sb, cin0, pm, ct = tc1(x_text, Wt, bt.reshape(1, H), bat3, did2,
                           cpp, ctp, prop_emb, dom_emb)

    s0 = agg_call(cin0, src_s, dst_s, off16, zrs)
    relu0, cin1 = tc2(s0.reshape(R, NP, H), ct, cin0, pm,
                      W0, root0, b0.reshape(1, H))

    s1 = agg_call(cin1, src_s, dst_s, off16, zrs)
    relu1, _ = tc2(s1.reshape(R, NP, H), ct, cin1, pm,
                   W1, root1, b1.reshape(1, H))

    return tc3(sb, relu1, Ws, bs.reshape(1, H), Wst, bst.reshape(1, H),
               Wg1[:H], Wg1[H:], bg1.reshape(1, H), Wg2, bg2.reshape(1, H))


def kernel(x_text, edge_index, edge_type, batch, domain_ids, edge_property_id,
           Wt, bt, dom_emb, prop_emb, W0, root0, b0, W1, root1, b1,
           Ws, bs, Wst, bst, Wg1, bg1, Wg2, bg2):
    return _run(False, False,
                x_text, edge_index, edge_type, batch, domain_ids,
                edge_property_id, Wt, bt, dom_emb, prop_emb,
                W0, root0, b0, W1, root1, b1,
                Ws, bs, Wst, bst, Wg1, bg1, Wg2, bg2)
